# Initial kernel scaffold; baseline (speedup 1.0000x reference)
import jax
import jax.numpy as jnp
from jax.experimental import pallas as pl


def kernel(query, keys):
    raise NotImplementedError("write your pallas kernel here")


# speedup vs baseline: 2.4827x; 2.4827x over previous
import functools

import jax
import jax.numpy as jnp
from jax.experimental import pallas as pl
import jax.experimental.pallas.tpu as pltpu

MEM = 512
DIM = 512
BS = 8
HW = 1024
N = BS * HW
PJ = 2
PB = HW // PJ
HB = 32 // PJ
INV_T = 10.0


def _passA_kernel(q_ref, k_ref, uq_ref, sm_ref, colsum_ref, colmax_ref,
                  rowsum_ref, g0_ref, emax_ref, sep_ref, comp_ref):
    b = pl.program_id(0)
    j = pl.program_id(1)
    first = jnp.logical_and(b == 0, j == 0)

    qb = q_ref[0].reshape(DIM, PB)
    kb = k_ref[...]

    qnorm2 = jnp.sum(qb * qb, axis=0)
    qinv = 1.0 / jnp.maximum(jnp.sqrt(qnorm2), 1e-12)
    qn = qb * qinv[None, :]
    qn2 = qnorm2 * qinv * qinv

    knorm2 = jnp.sum(kb * kb, axis=1)
    kinv = 1.0 / jnp.maximum(jnp.sqrt(knorm2), 1e-12)
    mn = kb * kinv[:, None]
    keysum = jnp.sum(kb, axis=1)

    score = jax.lax.dot_general(
        qn, mn, (((0,), (1,)), ((), ())),
        preferred_element_type=jnp.float32)

    e = jnp.exp(score * INV_T)
    rowsum = jnp.sum(e, axis=1)
    sm = e * (1.0 / rowsum)[:, None]
    sm_ref[...] = sm

    csum = jnp.sum(e, axis=0)[None, :]
    cmax = jnp.max(e, axis=0)[None, :]

    @pl.when(first)
    def _():
        colsum_ref[...] = csum
        colmax_ref[...] = cmax

    @pl.when(jnp.logical_not(first))
    def _():
        colsum_ref[...] += csum
        colmax_ref[...] = jnp.maximum(colmax_ref[...], cmax)

    s0 = jnp.max(score, axis=1)
    iota_m = jax.lax.broadcasted_iota(jnp.int32, (PB, MEM), 1)
    g0 = jnp.argmax(score, axis=1).astype(jnp.int32)
    onehot0 = (iota_m == g0[:, None])
    masked = jnp.where(onehot0, -jnp.inf, score)
    s1 = jnp.max(masked, axis=1)
    g1 = jnp.argmax(masked, axis=1).astype(jnp.int32)
    onehot1 = (iota_m == g1[:, None])

    ks0 = jnp.sum(jnp.where(onehot0, keysum[None, :], 0.0), axis=1)
    ks1 = jnp.sum(jnp.where(onehot1, keysum[None, :], 0.0), axis=1)
    kn20 = jnp.sum(jnp.where(onehot0, knorm2[None, :], 0.0), axis=1)
    kn21 = jnp.sum(jnp.where(onehot1, knorm2[None, :], 0.0), axis=1)
    kn0 = jnp.sqrt(kn20)
    kn1 = jnp.sqrt(kn21)

    rsq = jnp.sum(qn, axis=0)

    base0 = qn2 + kn20 - 2.0 * kn0 * s0
    base1 = qn2 + kn21 - 2.0 * kn1 * s1
    eps_d = DIM * 1e-12
    dp = jnp.sqrt(jnp.maximum(base0 + 2e-6 * (rsq - ks0) + eps_d, 0.0))
    dn = jnp.sqrt(jnp.maximum(base1 + 2e-6 * (rsq - ks1) + eps_d, 0.0))
    sep = jnp.sum(jnp.maximum(dp - dn + 1.0, 0.0)).reshape(1, 1)
    comp = jnp.sum(base0).reshape(1, 1)

    @pl.when(first)
    def _():
        sep_ref[...] = sep
        comp_ref[...] = comp

    @pl.when(jnp.logical_not(first))
    def _():
        sep_ref[...] += sep
        comp_ref[...] += comp

    rowsum_ref[...] = rowsum.reshape(1, 1, PB)
    g0_ref[...] = g0.reshape(1, 1, PB)
    emax_ref[...] = jnp.exp(s0 * INV_T).reshape(1, 1, PB)

    cmT = jax.lax.dot_general(
        kb, sm, (((0,), (1,)), ((), ())),
        preferred_element_type=jnp.float32)
    uq_ref[0, :DIM] = qn.reshape(DIM, HB, 32)
    uq_ref[0, DIM:] = cmT.reshape(DIM, HB, 32)


def _passB_kernel(sm_ref, rowsum_ref, colsum_ref, colmax_ref, g0_ref,
                  emax_ref, qn_ref, k_ref, sq_ref, um_ref,
                  numer_ref, denom_ref):
    b = pl.program_id(0)
    j = pl.program_id(1)
    first = jnp.logical_and(b == 0, j == 0)
    last = jnp.logical_and(b == BS - 1, j == PJ - 1)

    colsum = colsum_ref[0, :]
    colmax = colmax_ref[0, :]
    g0 = g0_ref[0, 0, :]
    emax = emax_ref[0, 0, :]
    rowsum = rowsum_ref[0, 0, :]

    sm = sm_ref[...]
    sq_ref[...] = sm * rowsum[:, None] * (1.0 / colsum)[None, :]

    iota_m = jax.lax.broadcasted_iota(jnp.int32, (PB, MEM), 1)
    onehot = (iota_m == g0[:, None])
    cmax_g = jnp.sum(jnp.where(onehot, colmax[None, :], 0.0), axis=1)
    csum_g = jnp.sum(jnp.where(onehot, colsum[None, :], 0.0), axis=1)
    wts = emax / (cmax_g + 1e-8 * csum_g)

    w = jnp.where(onehot, wts[:, None], 0.0)
    qn = qn_ref[0].reshape(DIM, PB)
    part = jax.lax.dot_general(
        w, qn, (((0,), (1,)), ((), ())),
        preferred_element_type=jnp.float32)
    dpart = jnp.sum(w, axis=0)[None, :]

    @pl.when(first)
    def _():
        numer_ref[...] = part
        denom_ref[...] = dpart

    @pl.when(jnp.logical_not(first))
    def _():
        numer_ref[...] += part
        denom_ref[...] += dpart

    @pl.when(last)
    def _():
        den = denom_ref[0, :] + 1e-8
        qu = numer_ref[...] * (1.0 / den)[:, None]
        upd = qu + k_ref[...]
        unorm = jnp.sqrt(jnp.sum(upd * upd, axis=1))
        um_ref[...] = upd * (1.0 / jnp.maximum(unorm, 1e-12))[:, None]


@functools.partial(jax.jit, static_argnames=("interpret",))
def _run(query, keys, interpret=False):
    passA = pl.pallas_call(
        _passA_kernel,
        grid=(BS, PJ),
        in_specs=[
            pl.BlockSpec((1, DIM, HB, 32), lambda b, j: (b, 0, j, 0)),
            pl.BlockSpec((MEM, DIM), lambda b, j: (0, 0)),
        ],
        out_specs=[
            pl.BlockSpec((1, 2 * DIM, HB, 32), lambda b, j: (b, 0, j, 0)),
            pl.BlockSpec((PB, MEM), lambda b, j: (b * PJ + j, 0)),
            pl.BlockSpec((1, MEM), lambda b, j: (0, 0)),
            pl.BlockSpec((1, MEM), lambda b, j: (0, 0)),
            pl.BlockSpec((1, 1, PB), lambda b, j: (b, 0, j)),
            pl.BlockSpec((1, 1, PB), lambda b, j: (b, 0, j)),
            pl.BlockSpec((1, 1, PB), lambda b, j: (b, 0, j)),
            pl.BlockSpec((1, 1), lambda b, j: (0, 0)),
            pl.BlockSpec((1, 1), lambda b, j: (0, 0)),
        ],
        out_shape=[
            jax.ShapeDtypeStruct((BS, 2 * DIM, 32, 32), jnp.float32),
            jax.ShapeDtypeStruct((N, MEM), jnp.float32),
            jax.ShapeDtypeStruct((1, MEM), jnp.float32),
            jax.ShapeDtypeStruct((1, MEM), jnp.float32),
            jax.ShapeDtypeStruct((BS, 1, HW), jnp.float32),
            jax.ShapeDtypeStruct((BS, 1, HW), jnp.int32),
            jax.ShapeDtypeStruct((BS, 1, HW), jnp.float32),
            jax.ShapeDtypeStruct((1, 1), jnp.float32),
            jax.ShapeDtypeStruct((1, 1), jnp.float32),
        ],
        interpret=interpret,
    )
    uq, sm, colsum, colmax, rowsum, g0, emax, sep, comp = passA(query, keys)

    passB = pl.pallas_call(
        _passB_kernel,
        grid=(BS, PJ),
        in_specs=[
            pl.BlockSpec((PB, MEM), lambda b, j: (b * PJ + j, 0)),
            pl.BlockSpec((1, 1, PB), lambda b, j: (b, 0, j)),
            pl.BlockSpec((1, MEM), lambda b, j: (0, 0)),
            pl.BlockSpec((1, MEM), lambda b, j: (0, 0)),
            pl.BlockSpec((1, 1, PB), lambda b, j: (b, 0, j)),
            pl.BlockSpec((1, 1, PB), lambda b, j: (b, 0, j)),
            pl.BlockSpec((1, DIM, HB, 32), lambda b, j: (b, 0, j, 0)),
            pl.BlockSpec((MEM, DIM), lambda b, j: (0, 0)),
        ],
        out_specs=[
            pl.BlockSpec((PB, MEM), lambda b, j: (b * PJ + j, 0)),
            pl.BlockSpec((MEM, DIM), lambda b, j: (0, 0)),
        ],
        out_shape=[
            jax.ShapeDtypeStruct((N, MEM), jnp.float32),
            jax.ShapeDtypeStruct((MEM, DIM), jnp.float32),
        ],
        scratch_shapes=[
            pltpu.VMEM((MEM, DIM), jnp.float32),
            pltpu.VMEM((1, MEM), jnp.float32),
        ],
        interpret=interpret,
    )
    sq, um = passB(sm, rowsum, colsum, colmax, g0, emax, uq, keys)

    sep_s = (sep[0, 0] / N).astype(jnp.float32)
    comp_s = (comp[0, 0] / (N * DIM)).astype(jnp.float32)
    return uq, um, sq, sm, sep_s, comp_s


def kernel(query, keys):
    return _run(query, keys)


# speedup vs baseline: 5.0958x; 2.0526x over previous
import functools

import jax
import jax.numpy as jnp
from jax.experimental import pallas as pl
import jax.experimental.pallas.tpu as pltpu

MEM = 512
DIM = 512
BS = 8
HW = 1024
N = BS * HW
PJ = 2
PB = HW // PJ
INV_T = 10.0


def _passA_kernel(q_ref, k_ref, uq_ref, sm_ref, colsum_ref, colmax_ref,
                  rowsum_ref, g0_ref, emax_ref, sep_ref, comp_ref,
                  mn_ref, ks_ref, kn2_ref):
    b = pl.program_id(0)
    j = pl.program_id(1)
    first = jnp.logical_and(b == 0, j == 0)

    @pl.when(first)
    def _():
        kb = k_ref[...]
        kn2 = jnp.sum(kb * kb, axis=1)
        kinv = 1.0 / jnp.maximum(jnp.sqrt(kn2), 1e-12)
        mn_ref[...] = kb * kinv[:, None]
        ks_ref[...] = jnp.sum(kb, axis=1).reshape(1, MEM)
        kn2_ref[...] = kn2.reshape(1, MEM)

    mn = mn_ref[...]
    keysum = ks_ref[0, :]
    knorm2 = kn2_ref[0, :]

    qb = q_ref[0]
    qnorm2 = jnp.sum(qb * qb, axis=0)
    qinv = 1.0 / jnp.maximum(jnp.sqrt(qnorm2), 1e-12)
    qn = qb * qinv[None, :]
    qn2 = qnorm2 * qinv * qinv

    score = jax.lax.dot_general(
        qn, mn, (((0,), (1,)), ((), ())),
        preferred_element_type=jnp.float32)

    e = jnp.exp(score * INV_T)
    rowsum = jnp.sum(e, axis=1)
    sm = e * (1.0 / rowsum)[:, None]
    sm_ref[...] = sm

    csum = jnp.sum(e, axis=0)[None, :]
    cmax = jnp.max(e, axis=0)[None, :]

    @pl.when(first)
    def _():
        colsum_ref[...] = csum
        colmax_ref[...] = cmax

    @pl.when(jnp.logical_not(first))
    def _():
        colsum_ref[...] += csum
        colmax_ref[...] = jnp.maximum(colmax_ref[...], cmax)

    s0 = jnp.max(score, axis=1)
    iota_m = jax.lax.broadcasted_iota(jnp.int32, (PB, MEM), 1)
    g0 = jnp.argmax(score, axis=1).astype(jnp.int32)
    onehot0 = (iota_m == g0[:, None])
    masked = jnp.where(onehot0, -jnp.inf, score)
    s1 = jnp.max(masked, axis=1)
    g1 = jnp.argmax(masked, axis=1).astype(jnp.int32)
    onehot1 = (iota_m == g1[:, None])

    ks0 = jnp.sum(jnp.where(onehot0, keysum[None, :], 0.0), axis=1)
    ks1 = jnp.sum(jnp.where(onehot1, keysum[None, :], 0.0), axis=1)
    kn20 = jnp.sum(jnp.where(onehot0, knorm2[None, :], 0.0), axis=1)
    kn21 = jnp.sum(jnp.where(onehot1, knorm2[None, :], 0.0), axis=1)
    kn0 = jnp.sqrt(kn20)
    kn1 = jnp.sqrt(kn21)

    rsq = jnp.sum(qn, axis=0)

    base0 = qn2 + kn20 - 2.0 * kn0 * s0
    base1 = qn2 + kn21 - 2.0 * kn1 * s1
    eps_d = DIM * 1e-12
    dp = jnp.sqrt(jnp.maximum(base0 + 2e-6 * (rsq - ks0) + eps_d, 0.0))
    dn = jnp.sqrt(jnp.maximum(base1 + 2e-6 * (rsq - ks1) + eps_d, 0.0))
    sep = jnp.sum(jnp.maximum(dp - dn + 1.0, 0.0)).reshape(1, 1)
    comp = jnp.sum(base0).reshape(1, 1)

    @pl.when(first)
    def _():
        sep_ref[...] = sep
        comp_ref[...] = comp

    @pl.when(jnp.logical_not(first))
    def _():
        sep_ref[...] += sep
        comp_ref[...] += comp

    rowsum_ref[...] = rowsum.reshape(1, 1, PB)
    g0_ref[...] = g0.reshape(1, 1, PB)
    emax_ref[...] = jnp.exp(s0 * INV_T).reshape(1, 1, PB)

    cmT = jax.lax.dot_general(
        k_ref[...], sm, (((0,), (1,)), ((), ())),
        preferred_element_type=jnp.float32)
    uq_ref[0, :DIM, :] = qn
    uq_ref[0, DIM:, :] = cmT


def _passB_kernel(sm_ref, rowsum_ref, colsum_ref, colmax_ref, g0_ref,
                  emax_ref, qn_ref, k_ref, sq_ref, um_ref,
                  numer_ref, denom_ref):
    b = pl.program_id(0)
    j = pl.program_id(1)
    first = jnp.logical_and(b == 0, j == 0)
    last = jnp.logical_and(b == BS - 1, j == PJ - 1)

    colsum = colsum_ref[0, :]
    colmax = colmax_ref[0, :]
    g0 = g0_ref[0, 0, :]
    emax = emax_ref[0, 0, :]
    rowsum = rowsum_ref[0, 0, :]

    sm = sm_ref[...]
    sq_ref[...] = sm * rowsum[:, None] * (1.0 / colsum)[None, :]

    iota_m = jax.lax.broadcasted_iota(jnp.int32, (PB, MEM), 1)
    onehot = (iota_m == g0[:, None])
    cmax_g = jnp.sum(jnp.where(onehot, colmax[None, :], 0.0), axis=1)
    csum_g = jnp.sum(jnp.where(onehot, colsum[None, :], 0.0), axis=1)
    wts = emax / (cmax_g + 1e-8 * csum_g)

    w = jnp.where(onehot, wts[:, None], 0.0)
    qn = qn_ref[0]
    part = jax.lax.dot_general(
        w, qn, (((0,), (1,)), ((), ())),
        preferred_element_type=jnp.float32)
    dpart = jnp.sum(w, axis=0)[None, :]

    @pl.when(first)
    def _():
        numer_ref[...] = part
        denom_ref[...] = dpart

    @pl.when(jnp.logical_not(first))
    def _():
        numer_ref[...] += part
        denom_ref[...] += dpart

    @pl.when(last)
    def _():
        den = denom_ref[0, :] + 1e-8
        qu = numer_ref[...] * (1.0 / den)[:, None]
        upd = qu + k_ref[...]
        unorm = jnp.sqrt(jnp.sum(upd * upd, axis=1))
        um_ref[...] = upd * (1.0 / jnp.maximum(unorm, 1e-12))[:, None]


@functools.partial(jax.jit, static_argnames=("interpret",))
def _run(query, keys, interpret=False):
    query_r = query.reshape(BS, DIM, HW)

    passA = pl.pallas_call(
        _passA_kernel,
        grid=(BS, PJ),
        in_specs=[
            pl.BlockSpec((1, DIM, PB), lambda b, j: (b, 0, j)),
            pl.BlockSpec((MEM, DIM), lambda b, j: (0, 0)),
        ],
        out_specs=[
            pl.BlockSpec((1, 2 * DIM, PB), lambda b, j: (b, 0, j)),
            pl.BlockSpec((PB, MEM), lambda b, j: (b * PJ + j, 0)),
            pl.BlockSpec((1, MEM), lambda b, j: (0, 0)),
            pl.BlockSpec((1, MEM), lambda b, j: (0, 0)),
            pl.BlockSpec((1, 1, PB), lambda b, j: (b, 0, j)),
            pl.BlockSpec((1, 1, PB), lambda b, j: (b, 0, j)),
            pl.BlockSpec((1, 1, PB), lambda b, j: (b, 0, j)),
            pl.BlockSpec((1, 1), lambda b, j: (0, 0)),
            pl.BlockSpec((1, 1), lambda b, j: (0, 0)),
        ],
        out_shape=[
            jax.ShapeDtypeStruct((BS, 2 * DIM, HW), jnp.float32),
            jax.ShapeDtypeStruct((N, MEM), jnp.float32),
            jax.ShapeDtypeStruct((1, MEM), jnp.float32),
            jax.ShapeDtypeStruct((1, MEM), jnp.float32),
            jax.ShapeDtypeStruct((BS, 1, HW), jnp.float32),
            jax.ShapeDtypeStruct((BS, 1, HW), jnp.int32),
            jax.ShapeDtypeStruct((BS, 1, HW), jnp.float32),
            jax.ShapeDtypeStruct((1, 1), jnp.float32),
            jax.ShapeDtypeStruct((1, 1), jnp.float32),
        ],
        scratch_shapes=[
            pltpu.VMEM((MEM, DIM), jnp.float32),
            pltpu.VMEM((1, MEM), jnp.float32),
            pltpu.VMEM((1, MEM), jnp.float32),
        ],
        interpret=interpret,
    )
    uq_r, sm, colsum, colmax, rowsum, g0, emax, sep, comp = passA(
        query_r, keys)

    passB = pl.pallas_call(
        _passB_kernel,
        grid=(BS, PJ),
        in_specs=[
            pl.BlockSpec((PB, MEM), lambda b, j: (b * PJ + j, 0)),
            pl.BlockSpec((1, 1, PB), lambda b, j: (b, 0, j)),
            pl.BlockSpec((1, MEM), lambda b, j: (0, 0)),
            pl.BlockSpec((1, MEM), lambda b, j: (0, 0)),
            pl.BlockSpec((1, 1, PB), lambda b, j: (b, 0, j)),
            pl.BlockSpec((1, 1, PB), lambda b, j: (b, 0, j)),
            pl.BlockSpec((1, DIM, PB), lambda b, j: (b, 0, j)),
            pl.BlockSpec((MEM, DIM), lambda b, j: (0, 0)),
        ],
        out_specs=[
            pl.BlockSpec((PB, MEM), lambda b, j: (b * PJ + j, 0)),
            pl.BlockSpec((MEM, DIM), lambda b, j: (0, 0)),
        ],
        out_shape=[
            jax.ShapeDtypeStruct((N, MEM), jnp.float32),
            jax.ShapeDtypeStruct((MEM, DIM), jnp.float32),
        ],
        scratch_shapes=[
            pltpu.VMEM((MEM, DIM), jnp.float32),
            pltpu.VMEM((1, MEM), jnp.float32),
        ],
        interpret=interpret,
    )
    sq, um = passB(sm, rowsum, colsum, colmax, g0, emax, uq_r, keys)

    uq = uq_r.reshape(BS, 2 * DIM, 32, 32)
    sep_s = (sep[0, 0] / N).astype(jnp.float32)
    comp_s = (comp[0, 0] / (N * DIM)).astype(jnp.float32)
    return uq, um, sq, sm, sep_s, comp_s


def kernel(query, keys):
    return _run(query, keys)


# speedup vs baseline: 6.2056x; 1.2178x over previous
import functools

import jax
import jax.numpy as jnp
from jax.experimental import pallas as pl
import jax.experimental.pallas.tpu as pltpu

MEM = 512
DIM = 512
BS = 8
HW = 1024
N = BS * HW
PJ = 2
PB = HW // PJ
INV_T = 10.0


def _passA_kernel(q_ref, k_ref, uq_ref, sm_ref, colsum_ref, colmax_ref,
                  rowsum_ref, emax_ref, sep_ref, comp_ref,
                  mn_ref, pk_ref):
    b = pl.program_id(0)
    j = pl.program_id(1)
    first = jnp.logical_and(b == 0, j == 0)

    @pl.when(first)
    def _():
        kb = k_ref[...]
        kn2 = jnp.sum(kb * kb, axis=1)
        kinv = 1.0 / jnp.maximum(jnp.sqrt(kn2), 1e-12)
        mn_ref[...] = kb * kinv[:, None]
        ksum = jnp.sum(kb, axis=1)
        lane = jax.lax.broadcasted_iota(jnp.int32, (MEM, 128), 1)
        pk_ref[...] = jnp.where(
            lane == 0, ksum[:, None],
            jnp.where(lane == 1, kn2[:, None], 0.0))

    mn = mn_ref[...]

    qb = q_ref[0]
    qnorm2 = jnp.sum(qb * qb, axis=0)
    qinv = 1.0 / jnp.maximum(jnp.sqrt(qnorm2), 1e-12)
    qn = qb * qinv[None, :]
    qn2 = qnorm2 * qinv * qinv

    score = jax.lax.dot_general(
        qn, mn, (((0,), (1,)), ((), ())),
        preferred_element_type=jnp.float32)

    e = jnp.exp(score * INV_T)
    rowsum = jnp.sum(e, axis=1)
    sm = e * (1.0 / rowsum)[:, None]
    sm_ref[...] = sm

    csum = jnp.sum(e, axis=0)[None, :]
    cmax = jnp.max(e, axis=0)[None, :]

    @pl.when(first)
    def _():
        colsum_ref[...] = csum
        colmax_ref[...] = cmax

    @pl.when(jnp.logical_not(first))
    def _():
        colsum_ref[...] += csum
        colmax_ref[...] = jnp.maximum(colmax_ref[...], cmax)

    emax = jnp.max(e, axis=1)
    b0 = e == emax[:, None]
    f0 = jnp.where(b0, 1.0, 0.0)
    masked = jnp.where(b0, 0.0, e)
    e1 = jnp.max(masked, axis=1)
    f1 = jnp.where(masked == e1[:, None], 1.0, 0.0)

    pk = pk_ref[...]
    gat0 = jnp.dot(f0, pk, preferred_element_type=jnp.float32)
    gat1 = jnp.dot(f1, pk, preferred_element_type=jnp.float32)
    ks0 = gat0[:, 0]
    kn20 = gat0[:, 1]
    ks1 = gat1[:, 0]
    kn21 = gat1[:, 1]
    s0 = jnp.log(emax) * (1.0 / INV_T)
    s1 = jnp.log(e1) * (1.0 / INV_T)

    rsq = jnp.sum(qn, axis=0)

    base0 = qn2 + kn20 - 2.0 * jnp.sqrt(kn20) * s0
    base1 = qn2 + kn21 - 2.0 * jnp.sqrt(kn21) * s1
    eps_d = DIM * 1e-12
    dp = jnp.sqrt(jnp.maximum(base0 + 2e-6 * (rsq - ks0) + eps_d, 0.0))
    dn = jnp.sqrt(jnp.maximum(base1 + 2e-6 * (rsq - ks1) + eps_d, 0.0))
    sep = jnp.sum(jnp.maximum(dp - dn + 1.0, 0.0)).reshape(1, 1)
    comp = jnp.sum(base0).reshape(1, 1)

    @pl.when(first)
    def _():
        sep_ref[...] = sep
        comp_ref[...] = comp

    @pl.when(jnp.logical_not(first))
    def _():
        sep_ref[...] += sep
        comp_ref[...] += comp

    rowsum_ref[...] = rowsum.reshape(1, 1, PB)
    emax_ref[...] = emax.reshape(1, 1, PB)

    cmT = jax.lax.dot_general(
        k_ref[...], sm, (((0,), (1,)), ((), ())),
        preferred_element_type=jnp.float32)
    uq_ref[0, :DIM, :] = qn
    uq_ref[0, DIM:, :] = cmT


def _passB_kernel(sm_ref, rowsum_ref, colsum_ref, colmax_ref,
                  emax_ref, qn_ref, k_ref, sq_ref, um_ref,
                  numer_ref, denom_ref, pc_ref):
    b = pl.program_id(0)
    j = pl.program_id(1)
    first = jnp.logical_and(b == 0, j == 0)
    last = jnp.logical_and(b == BS - 1, j == PJ - 1)

    @pl.when(first)
    def _():
        lane = jax.lax.broadcasted_iota(jnp.int32, (MEM, 128), 1)
        pc_ref[...] = jnp.where(
            lane == 0, colmax_ref[0, :][:, None],
            jnp.where(lane == 1, colsum_ref[0, :][:, None], 0.0))

    emax = emax_ref[0, 0, :]
    rowsum = rowsum_ref[0, 0, :]

    sm = sm_ref[...]
    colsum = colsum_ref[0, :]
    sq_ref[...] = sm * rowsum[:, None] * (1.0 / colsum)[None, :]

    f0 = jnp.where(sm == jnp.max(sm, axis=1)[:, None], 1.0, 0.0)
    gat = jnp.dot(f0, pc_ref[...], preferred_element_type=jnp.float32)
    cmax_g = gat[:, 0]
    csum_g = gat[:, 1]
    wts = emax / (cmax_g + 1e-8 * csum_g)

    w = f0 * wts[:, None]
    qn = qn_ref[0]
    part = jax.lax.dot_general(
        w, qn, (((0,), (1,)), ((), ())),
        preferred_element_type=jnp.float32)
    dpart = jnp.sum(w, axis=0)[None, :]

    @pl.when(first)
    def _():
        numer_ref[...] = part
        denom_ref[...] = dpart

    @pl.when(jnp.logical_not(first))
    def _():
        numer_ref[...] += part
        denom_ref[...] += dpart

    @pl.when(last)
    def _():
        den = denom_ref[0, :] + 1e-8
        qu = numer_ref[...] * (1.0 / den)[:, None]
        upd = qu + k_ref[...]
        unorm = jnp.sqrt(jnp.sum(upd * upd, axis=1))
        um_ref[...] = upd * (1.0 / jnp.maximum(unorm, 1e-12))[:, None]


@functools.partial(jax.jit, static_argnames=("interpret",))
def _run(query, keys, interpret=False):
    query_r = query.reshape(BS, DIM, HW)

    passA = pl.pallas_call(
        _passA_kernel,
        grid=(BS, PJ),
        in_specs=[
            pl.BlockSpec((1, DIM, PB), lambda b, j: (b, 0, j)),
            pl.BlockSpec((MEM, DIM), lambda b, j: (0, 0)),
        ],
        out_specs=[
            pl.BlockSpec((1, 2 * DIM, PB), lambda b, j: (b, 0, j)),
            pl.BlockSpec((PB, MEM), lambda b, j: (b * PJ + j, 0)),
            pl.BlockSpec((1, MEM), lambda b, j: (0, 0)),
            pl.BlockSpec((1, MEM), lambda b, j: (0, 0)),
            pl.BlockSpec((1, 1, PB), lambda b, j: (b, 0, j)),
            pl.BlockSpec((1, 1, PB), lambda b, j: (b, 0, j)),
            pl.BlockSpec((1, 1), lambda b, j: (0, 0)),
            pl.BlockSpec((1, 1), lambda b, j: (0, 0)),
        ],
        out_shape=[
            jax.ShapeDtypeStruct((BS, 2 * DIM, HW), jnp.float32),
            jax.ShapeDtypeStruct((N, MEM), jnp.float32),
            jax.ShapeDtypeStruct((1, MEM), jnp.float32),
            jax.ShapeDtypeStruct((1, MEM), jnp.float32),
            jax.ShapeDtypeStruct((BS, 1, HW), jnp.float32),
            jax.ShapeDtypeStruct((BS, 1, HW), jnp.float32),
            jax.ShapeDtypeStruct((1, 1), jnp.float32),
            jax.ShapeDtypeStruct((1, 1), jnp.float32),
        ],
        scratch_shapes=[
            pltpu.VMEM((MEM, DIM), jnp.float32),
            pltpu.VMEM((MEM, 128), jnp.float32),
        ],
        interpret=interpret,
    )
    uq_r, sm, colsum, colmax, rowsum, emax, sep, comp = passA(query_r, keys)

    passB = pl.pallas_call(
        _passB_kernel,
        grid=(BS, PJ),
        in_specs=[
            pl.BlockSpec((PB, MEM), lambda b, j: (b * PJ + j, 0)),
            pl.BlockSpec((1, 1, PB), lambda b, j: (b, 0, j)),
            pl.BlockSpec((1, MEM), lambda b, j: (0, 0)),
            pl.BlockSpec((1, MEM), lambda b, j: (0, 0)),
            pl.BlockSpec((1, 1, PB), lambda b, j: (b, 0, j)),
            pl.BlockSpec((1, DIM, PB), lambda b, j: (b, 0, j)),
            pl.BlockSpec((MEM, DIM), lambda b, j: (0, 0)),
        ],
        out_specs=[
            pl.BlockSpec((PB, MEM), lambda b, j: (b * PJ + j, 0)),
            pl.BlockSpec((MEM, DIM), lambda b, j: (0, 0)),
        ],
        out_shape=[
            jax.ShapeDtypeStruct((N, MEM), jnp.float32),
            jax.ShapeDtypeStruct((MEM, DIM), jnp.float32),
        ],
        scratch_shapes=[
            pltpu.VMEM((MEM, DIM), jnp.float32),
            pltpu.VMEM((1, MEM), jnp.float32),
            pltpu.VMEM((MEM, 128), jnp.float32),
        ],
        interpret=interpret,
    )
    sq, um = passB(sm, rowsum, colsum, colmax, emax, uq_r, keys)

    uq = uq_r.reshape(BS, 2 * DIM, 32, 32)
    sep_s = (sep[0, 0] / N).astype(jnp.float32)
    comp_s = (comp[0, 0] / (N * DIM)).astype(jnp.float32)
    return uq, um, sq, sm, sep_s, comp_s


def kernel(query, keys):
    return _run(query, keys)


# speedup vs baseline: 7.6290x; 1.2294x over previous
import functools

import jax
import jax.numpy as jnp
from jax.experimental import pallas as pl
import jax.experimental.pallas.tpu as pltpu

MEM = 512
DIM = 512
BS = 8
HW = 1024
N = BS * HW
PJ = 2
PB = HW // PJ
NSTEP = BS * PJ
QSTEP = 4
QB = N // QSTEP
INV_T = 10.0


def _fused_kernel(q_ref, k_ref,
                  uq_ref, sm_ref, sq_ref, um_ref, sep_ref, comp_ref,
                  mn_ref, pk_ref, e_ref, colsum_ref, colmax_ref,
                  rawnum_ref, rawden_ref):
    s = pl.program_id(0)
    first = s == 0

    @pl.when(first)
    def _():
        kb = k_ref[...]
        kn2 = jnp.sum(kb * kb, axis=1)
        kinv = 1.0 / jnp.maximum(jnp.sqrt(kn2), 1e-12)
        mn_ref[...] = kb * kinv[:, None]
        ksum = jnp.sum(kb, axis=1)
        lane = jax.lax.broadcasted_iota(jnp.int32, (MEM, 128), 1)
        pk_ref[...] = jnp.where(
            lane == 0, ksum[:, None],
            jnp.where(lane == 1, kn2[:, None], 0.0))

    @pl.when(s < NSTEP)
    def _():
        mn = mn_ref[...]
        qb = q_ref[0]
        qnorm2 = jnp.sum(qb * qb, axis=0)
        qinv = 1.0 / jnp.maximum(jnp.sqrt(qnorm2), 1e-12)
        qn = qb * qinv[None, :]
        qn2 = qnorm2 * qinv * qinv

        score = jax.lax.dot_general(
            qn, mn, (((0,), (1,)), ((), ())),
            preferred_element_type=jnp.float32)

        e = jnp.exp(score * INV_T)
        rowsum = jnp.sum(e, axis=1)
        sm = e * (1.0 / rowsum)[:, None]
        sm_ref[...] = sm
        e_ref[pl.ds(s * PB, PB), :] = e

        csum = jnp.sum(e, axis=0)[None, :]
        cmax = jnp.max(e, axis=0)[None, :]

        emax = jnp.max(e, axis=1)
        b0 = e == emax[:, None]
        f0 = jnp.where(b0, 1.0, 0.0)
        masked = jnp.where(b0, 0.0, e)
        e1 = jnp.max(masked, axis=1)
        f1 = jnp.where(masked == e1[:, None], 1.0, 0.0)

        pk = pk_ref[...]
        gat0 = jnp.dot(f0, pk, preferred_element_type=jnp.float32)
        gat1 = jnp.dot(f1, pk, preferred_element_type=jnp.float32)
        ks0 = gat0[:, 0]
        kn20 = gat0[:, 1]
        ks1 = gat1[:, 0]
        kn21 = gat1[:, 1]
        s0 = jnp.log(emax) * (1.0 / INV_T)
        s1 = jnp.log(e1) * (1.0 / INV_T)

        rsq = jnp.sum(qn, axis=0)

        base0 = qn2 + kn20 - 2.0 * jnp.sqrt(kn20) * s0
        base1 = qn2 + kn21 - 2.0 * jnp.sqrt(kn21) * s1
        eps_d = DIM * 1e-12
        dp = jnp.sqrt(jnp.maximum(base0 + 2e-6 * (rsq - ks0) + eps_d, 0.0))
        dn = jnp.sqrt(jnp.maximum(base1 + 2e-6 * (rsq - ks1) + eps_d, 0.0))
        sep = jnp.sum(jnp.maximum(dp - dn + 1.0, 0.0)).reshape(1, 1)
        comp = jnp.sum(base0).reshape(1, 1)

        w0 = f0 * emax[:, None]
        part = jax.lax.dot_general(
            w0, qn, (((0,), (1,)), ((), ())),
            preferred_element_type=jnp.float32)
        dpart = jnp.sum(w0, axis=0)[None, :]

        @pl.when(first)
        def _():
            colsum_ref[...] = csum
            colmax_ref[...] = cmax
            rawnum_ref[...] = part
            rawden_ref[...] = dpart
            sep_ref[...] = sep
            comp_ref[...] = comp

        @pl.when(jnp.logical_not(first))
        def _():
            colsum_ref[...] += csum
            colmax_ref[...] = jnp.maximum(colmax_ref[...], cmax)
            rawnum_ref[...] += part
            rawden_ref[...] += dpart
            sep_ref[...] += sep
            comp_ref[...] += comp

        cmT = jax.lax.dot_general(
            k_ref[...], sm, (((0,), (1,)), ((), ())),
            preferred_element_type=jnp.float32)
        uq_ref[0, :DIM, :] = qn
        uq_ref[0, DIM:, :] = cmT

    @pl.when(s >= NSTEP)
    def _():
        i = s - NSTEP
        ec = e_ref[pl.ds(i * QB, QB), :]
        sq_ref[...] = ec * (1.0 / colsum_ref[0, :])[None, :]

        @pl.when(s == NSTEP)
        def _():
            den = rawden_ref[0, :] + 1e-8 * (
                colmax_ref[0, :] + 1e-8 * colsum_ref[0, :])
            qu = rawnum_ref[...] * (1.0 / den)[:, None]
            upd = qu + k_ref[...]
            unorm = jnp.sqrt(jnp.sum(upd * upd, axis=1))
            um_ref[...] = upd * (1.0 / jnp.maximum(unorm, 1e-12))[:, None]


@functools.partial(jax.jit, static_argnames=("interpret",))
def _run(query, keys, interpret=False):
    query_r = query.reshape(BS, DIM, HW)

    fused = pl.pallas_call(
        _fused_kernel,
        grid=(NSTEP + QSTEP,),
        in_specs=[
            pl.BlockSpec(
                (1, DIM, PB),
                lambda s: (jnp.minimum(s, NSTEP - 1) // PJ, 0,
                           jnp.minimum(s, NSTEP - 1) % PJ)),
            pl.BlockSpec((MEM, DIM), lambda s: (0, 0)),
        ],
        out_specs=[
            pl.BlockSpec(
                (1, 2 * DIM, PB),
                lambda s: (jnp.minimum(s, NSTEP - 1) // PJ, 0,
                           jnp.minimum(s, NSTEP - 1) % PJ)),
            pl.BlockSpec((PB, MEM), lambda s: (jnp.minimum(s, NSTEP - 1), 0)),
            pl.BlockSpec((QB, MEM), lambda s: (jnp.maximum(s - NSTEP, 0), 0)),
            pl.BlockSpec((MEM, DIM), lambda s: (0, 0)),
            pl.BlockSpec((1, 1), lambda s: (0, 0)),
            pl.BlockSpec((1, 1), lambda s: (0, 0)),
        ],
        out_shape=[
            jax.ShapeDtypeStruct((BS, 2 * DIM, HW), jnp.float32),
            jax.ShapeDtypeStruct((N, MEM), jnp.float32),
            jax.ShapeDtypeStruct((N, MEM), jnp.float32),
            jax.ShapeDtypeStruct((MEM, DIM), jnp.float32),
            jax.ShapeDtypeStruct((1, 1), jnp.float32),
            jax.ShapeDtypeStruct((1, 1), jnp.float32),
        ],
        scratch_shapes=[
            pltpu.VMEM((MEM, DIM), jnp.float32),
            pltpu.VMEM((MEM, 128), jnp.float32),
            pltpu.VMEM((N, MEM), jnp.float32),
            pltpu.VMEM((1, MEM), jnp.float32),
            pltpu.VMEM((1, MEM), jnp.float32),
            pltpu.VMEM((MEM, DIM), jnp.float32),
            pltpu.VMEM((1, MEM), jnp.float32),
        ],
        interpret=interpret,
    )
    uq_r, sm, sq, um, sep, comp = fused(query_r, keys)

    uq = uq_r.reshape(BS, 2 * DIM, 32, 32)
    sep_s = (sep[0, 0] / N).astype(jnp.float32)
    comp_s = (comp[0, 0] / (N * DIM)).astype(jnp.float32)
    return uq, um, sq, sm, sep_s, comp_s


def kernel(query, keys):
    return _run(query, keys)


# speedup vs baseline: 8.0823x; 1.0594x over previous
import functools

import jax
import jax.numpy as jnp
from jax.experimental import pallas as pl
import jax.experimental.pallas.tpu as pltpu

MEM = 512
DIM = 512
BS = 8
HW = 1024
N = BS * HW
PJ = 1
PB = HW // PJ
NSTEP = BS * PJ
QSTEP = 4
QB = N // QSTEP
INV_T = 10.0


def _fused_kernel(q_ref, k_ref,
                  uq_ref, sm_ref, sq_ref, um_ref, sep_ref, comp_ref,
                  mn_ref, pk_ref, e_ref, colsum_ref, colmax_ref,
                  rawnum_ref, rawden_ref):
    s = pl.program_id(0)
    first = s == 0

    @pl.when(first)
    def _():
        kb = k_ref[...]
        kn2 = jnp.sum(kb * kb, axis=1)
        kinv = 1.0 / jnp.maximum(jnp.sqrt(kn2), 1e-12)
        mn_ref[...] = kb * kinv[:, None]
        ksum = jnp.sum(kb, axis=1)
        lane = jax.lax.broadcasted_iota(jnp.int32, (MEM, 128), 1)
        pk_ref[...] = jnp.where(
            lane == 0, ksum[:, None],
            jnp.where(lane == 1, kn2[:, None], 0.0))

    @pl.when(s < NSTEP)
    def _():
        mn = mn_ref[...]
        qb = q_ref[0]
        qnorm2 = jnp.sum(qb * qb, axis=0)
        qinv = 1.0 / jnp.maximum(jnp.sqrt(qnorm2), 1e-12)
        qn = qb * qinv[None, :]
        qn2 = qnorm2 * qinv * qinv

        score = jax.lax.dot_general(
            qn, mn, (((0,), (1,)), ((), ())),
            preferred_element_type=jnp.float32)

        e = jnp.exp(score * INV_T)
        rowsum = jnp.sum(e, axis=1)
        sm = e * (1.0 / rowsum)[:, None]
        sm_ref[...] = sm
        e_ref[pl.ds(s * PB, PB), :] = e

        csum = jnp.sum(e, axis=0)[None, :]
        cmax = jnp.max(e, axis=0)[None, :]

        emax = jnp.max(e, axis=1)
        b0 = e == emax[:, None]
        f0 = jnp.where(b0, 1.0, 0.0)
        masked = jnp.where(b0, 0.0, e)
        e1 = jnp.max(masked, axis=1)
        f1 = jnp.where(masked == e1[:, None], 1.0, 0.0)

        pk = pk_ref[...]
        gat0 = jnp.dot(f0, pk, preferred_element_type=jnp.float32)
        gat1 = jnp.dot(f1, pk, preferred_element_type=jnp.float32)
        ks0 = gat0[:, 0]
        kn20 = gat0[:, 1]
        ks1 = gat1[:, 0]
        kn21 = gat1[:, 1]
        s0 = jnp.log(emax) * (1.0 / INV_T)
        s1 = jnp.log(e1) * (1.0 / INV_T)

        rsq = jnp.sum(qn, axis=0)

        base0 = qn2 + kn20 - 2.0 * jnp.sqrt(kn20) * s0
        base1 = qn2 + kn21 - 2.0 * jnp.sqrt(kn21) * s1
        eps_d = DIM * 1e-12
        dp = jnp.sqrt(jnp.maximum(base0 + 2e-6 * (rsq - ks0) + eps_d, 0.0))
        dn = jnp.sqrt(jnp.maximum(base1 + 2e-6 * (rsq - ks1) + eps_d, 0.0))
        sep = jnp.sum(jnp.maximum(dp - dn + 1.0, 0.0)).reshape(1, 1)
        comp = jnp.sum(base0).reshape(1, 1)

        w0 = f0 * emax[:, None]
        part = jax.lax.dot_general(
            w0, qn, (((0,), (1,)), ((), ())),
            preferred_element_type=jnp.float32)
        dpart = jnp.sum(w0, axis=0)[None, :]

        @pl.when(first)
        def _():
            colsum_ref[...] = csum
            colmax_ref[...] = cmax
            rawnum_ref[...] = part
            rawden_ref[...] = dpart
            sep_ref[...] = sep
            comp_ref[...] = comp

        @pl.when(jnp.logical_not(first))
        def _():
            colsum_ref[...] += csum
            colmax_ref[...] = jnp.maximum(colmax_ref[...], cmax)
            rawnum_ref[...] += part
            rawden_ref[...] += dpart
            sep_ref[...] += sep
            comp_ref[...] += comp

        cmT = jax.lax.dot_general(
            k_ref[...], sm, (((0,), (1,)), ((), ())),
            preferred_element_type=jnp.float32)
        uq_ref[0, :DIM, :] = qn
        uq_ref[0, DIM:, :] = cmT

    @pl.when(s >= NSTEP)
    def _():
        i = s - NSTEP
        ec = e_ref[pl.ds(i * QB, QB), :]
        sq_ref[...] = ec * (1.0 / colsum_ref[0, :])[None, :]

        @pl.when(s == NSTEP)
        def _():
            den = rawden_ref[0, :] + 1e-8 * (
                colmax_ref[0, :] + 1e-8 * colsum_ref[0, :])
            qu = rawnum_ref[...] * (1.0 / den)[:, None]
            upd = qu + k_ref[...]
            unorm = jnp.sqrt(jnp.sum(upd * upd, axis=1))
            um_ref[...] = upd * (1.0 / jnp.maximum(unorm, 1e-12))[:, None]


@functools.partial(jax.jit, static_argnames=("interpret",))
def _run(query, keys, interpret=False):
    query_r = query.reshape(BS, DIM, HW)

    fused = pl.pallas_call(
        _fused_kernel,
        grid=(NSTEP + QSTEP,),
        in_specs=[
            pl.BlockSpec(
                (1, DIM, PB),
                lambda s: (jnp.minimum(s, NSTEP - 1) // PJ, 0,
                           jnp.minimum(s, NSTEP - 1) % PJ)),
            pl.BlockSpec((MEM, DIM), lambda s: (0, 0)),
        ],
        out_specs=[
            pl.BlockSpec(
                (1, 2 * DIM, PB),
                lambda s: (jnp.minimum(s, NSTEP - 1) // PJ, 0,
                           jnp.minimum(s, NSTEP - 1) % PJ)),
            pl.BlockSpec((PB, MEM), lambda s: (jnp.minimum(s, NSTEP - 1), 0)),
            pl.BlockSpec((QB, MEM), lambda s: (jnp.maximum(s - NSTEP, 0), 0)),
            pl.BlockSpec((MEM, DIM), lambda s: (0, 0)),
            pl.BlockSpec((1, 1), lambda s: (0, 0)),
            pl.BlockSpec((1, 1), lambda s: (0, 0)),
        ],
        out_shape=[
            jax.ShapeDtypeStruct((BS, 2 * DIM, HW), jnp.float32),
            jax.ShapeDtypeStruct((N, MEM), jnp.float32),
            jax.ShapeDtypeStruct((N, MEM), jnp.float32),
            jax.ShapeDtypeStruct((MEM, DIM), jnp.float32),
            jax.ShapeDtypeStruct((1, 1), jnp.float32),
            jax.ShapeDtypeStruct((1, 1), jnp.float32),
        ],
        scratch_shapes=[
            pltpu.VMEM((MEM, DIM), jnp.float32),
            pltpu.VMEM((MEM, 128), jnp.float32),
            pltpu.VMEM((N, MEM), jnp.float32),
            pltpu.VMEM((1, MEM), jnp.float32),
            pltpu.VMEM((1, MEM), jnp.float32),
            pltpu.VMEM((MEM, DIM), jnp.float32),
            pltpu.VMEM((1, MEM), jnp.float32),
        ],
        interpret=interpret,
    )
    uq_r, sm, sq, um, sep, comp = fused(query_r, keys)

    uq = uq_r.reshape(BS, 2 * DIM, 32, 32)
    sep_s = (sep[0, 0] / N).astype(jnp.float32)
    comp_s = (comp[0, 0] / (N * DIM)).astype(jnp.float32)
    return uq, um, sq, sm, sep_s, comp_s


def kernel(query, keys):
    return _run(query, keys)
